# baseline (speedup 1.0000x reference)
import functools

import jax
import jax.numpy as jnp
from jax import lax
from jax.experimental import pallas as pl
from jax.experimental.pallas import tpu as pltpu
from jax.experimental.pallas import tpu_sc as plsc

D = 4096
HALF = D // 2
QUAD = D // 4
NC = 1
NS = 8
NW = NC * NS
PASS_W = HALF // NW
ROT_W = QUAD // NW
LANES = 16

TWO_PI = 6.283185307179586
INV_TWO_PI = 0.15915494309189535

_SIN_COEFFS = (
    1.0 / 6227020800.0,
    -1.0 / 39916800.0,
    1.0 / 362880.0,
    -1.0 / 5040.0,
    1.0 / 120.0,
    -1.0 / 6.0,
    1.0,
)
_COS_COEFFS = (
    -1.0 / 87178291200.0,
    1.0 / 479001600.0,
    -1.0 / 3628800.0,
    1.0 / 40320.0,
    -1.0 / 720.0,
    1.0 / 24.0,
    -0.5,
    1.0,
)


def _cry_body(x_hbm, ang_hbm, out_hbm, ang_v, xp_v, xj_v, xk_v, yj_v, yk_v,
              a_sem, jk_sem, p_sem, out_sem):
    wid = lax.axis_index("s") * NC + lax.axis_index("c")

    pbase = wid * PASS_W
    jbase = HALF + wid * ROT_W
    kbase = HALF + QUAD + wid * ROT_W

    cp_j = pltpu.make_async_copy(
        x_hbm.at[pl.ds(jbase, ROT_W)], xj_v, jk_sem)
    cp_k = pltpu.make_async_copy(
        x_hbm.at[pl.ds(kbase, ROT_W)], xk_v, jk_sem)
    cp_a = pltpu.make_async_copy(ang_hbm, ang_v.at[pl.ds(0, 1)], a_sem)
    cp_p = pltpu.make_async_copy(
        x_hbm.at[pl.ds(pbase, PASS_W)], xp_v, p_sem)
    cp_j.start()
    cp_k.start()
    cp_a.start()
    cp_p.start()
    cp_a.wait()

    zeros16 = lax.iota(jnp.int32, LANES) * 0
    dnums = lax.GatherDimensionNumbers(
        offset_dims=(), collapsed_slice_dims=(0,), start_index_map=(0,))
    ang_bcast = lax.gather(
        ang_v[...], zeros16[:, None], dnums, (1,),
        mode=lax.GatherScatterMode.PROMISE_IN_BOUNDS)
    theta = ang_bcast * 0.5
    t = theta * INV_TWO_PI
    n = (t + jnp.where(t >= 0.0, 0.5, -0.5)).astype(jnp.int32)
    r = theta - n.astype(jnp.float32) * TWO_PI
    r2 = r * r
    s = jnp.float32(_SIN_COEFFS[0])
    for coef in _SIN_COEFFS[1:]:
        s = s * r2 + coef
    s = s * r
    c = jnp.float32(_COS_COEFFS[0])
    for coef in _COS_COEFFS[1:]:
        c = c * r2 + coef

    cp_j.wait()
    cp_k.wait()
    for v in range(ROT_W // LANES):
        sl = pl.ds(v * LANES, LANES)
        xj = xj_v[sl]
        xk = xk_v[sl]
        yj_v[sl] = c * xj - s * xk
        yk_v[sl] = c * xk - s * xj

    cp_jo = pltpu.make_async_copy(yj_v, out_hbm.at[pl.ds(jbase, ROT_W)], out_sem)
    cp_ko = pltpu.make_async_copy(yk_v, out_hbm.at[pl.ds(kbase, ROT_W)], out_sem)
    cp_jo.start()
    cp_ko.start()
    cp_p.wait()
    cp_po = pltpu.make_async_copy(
        xp_v, out_hbm.at[pl.ds(pbase, PASS_W)], out_sem)
    cp_po.start()
    cp_jo.wait()
    cp_ko.wait()
    cp_po.wait()


@jax.jit
def _cry_real(xf, ang):
    mesh = plsc.VectorSubcoreMesh(
        core_axis_name="c", subcore_axis_name="s", num_cores=NC,
        num_subcores=NS)
    return pl.kernel(
        _cry_body,
        out_type=jax.ShapeDtypeStruct((D,), jnp.float32),
        mesh=mesh,
        scratch_types=[
            pltpu.VMEM((LANES,), jnp.float32),
            pltpu.VMEM((PASS_W,), jnp.float32),
            pltpu.VMEM((ROT_W,), jnp.float32),
            pltpu.VMEM((ROT_W,), jnp.float32),
            pltpu.VMEM((ROT_W,), jnp.float32),
            pltpu.VMEM((ROT_W,), jnp.float32),
            pltpu.SemaphoreType.DMA,
            pltpu.SemaphoreType.DMA,
            pltpu.SemaphoreType.DMA,
            pltpu.SemaphoreType.DMA,
        ],
    )(xf, ang)


def kernel(x, angle):
    xf = x.reshape(D)
    real = _cry_real(xf, angle.reshape(1))
    return real.astype(jnp.complex64).reshape(D, 1)


# speedup vs baseline: 1.0087x; 1.0087x over previous
import functools

import jax
import jax.numpy as jnp
from jax import lax
from jax.experimental import pallas as pl
from jax.experimental.pallas import tpu as pltpu
from jax.experimental.pallas import tpu_sc as plsc

D = 4096
HALF = D // 2
QUAD = D // 4
NC = 1
NW = NC * 16
PASS_W = HALF // NW
ROT_W = QUAD // NW
LANES = 16

TWO_PI = 6.283185307179586
INV_TWO_PI = 0.15915494309189535

_SIN_COEFFS = (
    1.0 / 6227020800.0,
    -1.0 / 39916800.0,
    1.0 / 362880.0,
    -1.0 / 5040.0,
    1.0 / 120.0,
    -1.0 / 6.0,
    1.0,
)
_COS_COEFFS = (
    -1.0 / 87178291200.0,
    1.0 / 479001600.0,
    -1.0 / 3628800.0,
    1.0 / 40320.0,
    -1.0 / 720.0,
    1.0 / 24.0,
    -0.5,
    1.0,
)


def _cry_body(x_hbm, ang_hbm, out_hbm, ang_v, xp_v, xj_v, xk_v, yj_v, yk_v,
              a_sem, jk_sem, p_sem, out_sem):
    wid = lax.axis_index("s") * NC + lax.axis_index("c")

    pbase = wid * PASS_W
    jbase = HALF + wid * ROT_W
    kbase = HALF + QUAD + wid * ROT_W

    cp_j = pltpu.make_async_copy(
        x_hbm.at[pl.ds(jbase, ROT_W)], xj_v, jk_sem)
    cp_k = pltpu.make_async_copy(
        x_hbm.at[pl.ds(kbase, ROT_W)], xk_v, jk_sem)
    cp_a = pltpu.make_async_copy(ang_hbm, ang_v.at[pl.ds(0, 1)], a_sem)
    cp_p = pltpu.make_async_copy(
        x_hbm.at[pl.ds(pbase, PASS_W)], xp_v, p_sem)
    cp_j.start()
    cp_k.start()
    cp_a.start()
    cp_p.start()
    cp_a.wait()

    zeros16 = lax.iota(jnp.int32, LANES) * 0
    dnums = lax.GatherDimensionNumbers(
        offset_dims=(), collapsed_slice_dims=(0,), start_index_map=(0,))
    ang_bcast = lax.gather(
        ang_v[...], zeros16[:, None], dnums, (1,),
        mode=lax.GatherScatterMode.PROMISE_IN_BOUNDS)
    theta = ang_bcast * 0.5
    t = theta * INV_TWO_PI
    n = (t + jnp.where(t >= 0.0, 0.5, -0.5)).astype(jnp.int32)
    r = theta - n.astype(jnp.float32) * TWO_PI
    r2 = r * r
    s = jnp.float32(_SIN_COEFFS[0])
    for coef in _SIN_COEFFS[1:]:
        s = s * r2 + coef
    s = s * r
    c = jnp.float32(_COS_COEFFS[0])
    for coef in _COS_COEFFS[1:]:
        c = c * r2 + coef

    cp_j.wait()
    cp_k.wait()
    for v in range(ROT_W // LANES):
        sl = pl.ds(v * LANES, LANES)
        xj = xj_v[sl]
        xk = xk_v[sl]
        yj_v[sl] = c * xj - s * xk
        yk_v[sl] = c * xk - s * xj

    cp_jo = pltpu.make_async_copy(yj_v, out_hbm.at[pl.ds(jbase, ROT_W)], out_sem)
    cp_ko = pltpu.make_async_copy(yk_v, out_hbm.at[pl.ds(kbase, ROT_W)], out_sem)
    cp_jo.start()
    cp_ko.start()
    cp_p.wait()
    cp_po = pltpu.make_async_copy(
        xp_v, out_hbm.at[pl.ds(pbase, PASS_W)], out_sem)
    cp_po.start()
    cp_jo.wait()
    cp_ko.wait()
    cp_po.wait()


@jax.jit
def _cry_real(xf, ang):
    mesh = plsc.VectorSubcoreMesh(
        core_axis_name="c", subcore_axis_name="s", num_cores=NC)
    return pl.kernel(
        _cry_body,
        out_type=jax.ShapeDtypeStruct((D,), jnp.float32),
        mesh=mesh,
        scratch_types=[
            pltpu.VMEM((LANES,), jnp.float32),
            pltpu.VMEM((PASS_W,), jnp.float32),
            pltpu.VMEM((ROT_W,), jnp.float32),
            pltpu.VMEM((ROT_W,), jnp.float32),
            pltpu.VMEM((ROT_W,), jnp.float32),
            pltpu.VMEM((ROT_W,), jnp.float32),
            pltpu.SemaphoreType.DMA,
            pltpu.SemaphoreType.DMA,
            pltpu.SemaphoreType.DMA,
            pltpu.SemaphoreType.DMA,
        ],
    )(xf, ang)


def kernel(x, angle):
    xf = x.reshape(D)
    real = _cry_real(xf, angle.reshape(1))
    return real.astype(jnp.complex64).reshape(D, 1)
